# baseline (speedup 1.0000x reference)
import functools

import jax
import jax.numpy as jnp
from jax import lax
from jax.experimental import pallas as pl
from jax.experimental.pallas import tpu as pltpu
from jax.experimental.pallas import tpu_sc as plsc

_V = 100000
_D = 64
_B = 16384
_R = 3 * _B
_NC, _NS, _L = 2, 16, 16
_NW = _NC * _NS
_RPW = _R // _NW
_C = 64
_NCH = _RPW // _C

_VP = 102400
_TBLK = 12800


def _tc_transpose_table(et):
    def body(inr, outr):
        outr[:, 0:_D] = jnp.transpose(inr[...])

    return pl.pallas_call(
        body,
        grid=(_VP // _TBLK,),
        in_specs=[pl.BlockSpec((_D, _TBLK), lambda i: (0, i))],
        out_specs=pl.BlockSpec((_TBLK, 2 * _D), lambda i: (i, 0)),
        out_shape=jax.ShapeDtypeStruct((_VP, 2 * _D), jnp.float32),
    )(et)


def _make_sc_encode(nf, mode):
    scratch = [
        pltpu.VMEM((_C,), jnp.int32),
        pltpu.VMEM((nf, _C), jnp.int32),
        pltpu.VMEM((_C + _L,), jnp.int32),
        pltpu.VMEM((nf, _C, 2 * _D), jnp.float32),
        pltpu.VMEM((_C // 2, 2 * _D), jnp.float32),
    ]
    if mode == "A":
        scratch.append(pltpu.VMEM((3 * _D,), jnp.float32))
    scratch.append(pltpu.SemaphoreType.DMA)

    @functools.partial(
        pl.kernel,
        mesh=plsc.VectorSubcoreMesh(core_axis_name="c", subcore_axis_name="s"),
        out_type=jax.ShapeDtypeStruct((_R // 2, 2 * _D), jnp.float32),
        compiler_params=pltpu.CompilerParams(use_tc_tiling_on_sc=False),
        scratch_types=scratch,
    )
    def enc(*args):
        xc0 = args[0]
        xcs = args[1:1 + nf]
        idx_all = args[1 + nf]
        if mode == "A":
            e0 = args[2 + nf]
            embs = args[3 + nf:3 + 2 * nf]
            hout, idx_v, cols, tlist, rbuf, hloc, emb0v, sem = args[3 + 2 * nf:]
        else:
            embs = args[2 + nf:2 + 2 * nf]
            hout, idx_v, cols, tlist, rbuf, hloc, sem = args[2 + 2 * nf:]
        wid = lax.axis_index("s") * _NC + lax.axis_index("c")
        if mode == "A":
            pltpu.sync_copy(e0, emb0v)
            e0sl = [[emb0v[pl.ds(t * _D + dv * _L, _L)]
                     for dv in range(_D // _L)] for t in range(3)]
        base_w = wid * _RPW

        def chunk(ci, carry):
            base = base_w + ci * _C
            pltpu.sync_copy(idx_all.at[pl.ds(base, _C)], idx_v)
            xcps = [pltpu.async_copy(xc0.at[idx_v], tlist.at[pl.ds(0, _C)], sem)]
            xcps += [pltpu.async_copy(xcs[f].at[idx_v], cols.at[f], sem)
                     for f in range(nf)]
            for cp in xcps:
                cp.wait()
            cps = [pltpu.async_copy(embs[f].at[cols.at[f]], rbuf.at[f], sem)
                   for f in range(nf)]
            for cp in cps:
                cp.wait()

            def row(r, c2):
                t = tlist[pl.ds(r, _L)][0]
                hrow = r >> 1
                hcol = (r & 1) * _D
                if mode == "A":
                    s0 = jnp.where(t == 0, 1.0, 0.0)
                    s1 = jnp.where(t == 1, 1.0, 0.0)
                    s2 = jnp.where(t == 2, 1.0, 0.0)
                    b0 = lax.broadcast(s0, (_L,))
                    b1 = lax.broadcast(s1, (_L,))
                    b2 = lax.broadcast(s2, (_L,))
                    for dv in range(_D // _L):
                        sl = pl.ds(dv * _L, _L)
                        h0 = (e0sl[0][dv] * b0 + e0sl[1][dv] * b1
                              + e0sl[2][dv] * b2)
                        a = (rbuf[0, r, sl] + rbuf[1, r, sl]
                             + rbuf[2, r, sl] + rbuf[3, r, sl])
                        hloc[hrow, pl.ds(hcol + dv * _L, _L)] = h0 + a * b0
                else:
                    s1 = jnp.where(t == 1, 1.0, 0.0)
                    s2 = jnp.where(t == 2, 1.0, 0.0)
                    b1 = lax.broadcast(s1, (_L,))
                    b2 = lax.broadcast(s2, (_L,))
                    for dv in range(_D // _L):
                        sl = pl.ds(dv * _L, _L)
                        b = rbuf[0, r, sl]
                        c = (rbuf[1, r, sl] + rbuf[2, r, sl]
                             + rbuf[3, r, sl] + rbuf[4, r, sl])
                        hloc[hrow, pl.ds(hcol + dv * _L, _L)] = b * b1 + c * b2
                return c2

            lax.fori_loop(0, _C, row, 0)
            pltpu.sync_copy(hloc, hout.at[pl.ds(base // 2, _C // 2)])
            return carry

        lax.fori_loop(0, _NCH, chunk, 0)

    return enc


_sc_encode_a = _make_sc_encode(4, "A")
_sc_encode_b = _make_sc_encode(5, "B")


_BLK2 = 1024
_NB2 = (_B // 2) // _BLK2


def _tc_epilogue(h2a, h2b, bm2, ew, eb, ow, ob):
    def body(hsa, hpa, hna, hsb, hpb, hnb, bmr, ewr, ebr, owr, obr,
             opos, oneg):
        w = owr[...]
        u = jnp.sum(w.T * ewr[...], axis=0, keepdims=True)
        const = jnp.sum(ebr[...] * w[0]) + obr[...][0]
        u2 = jnp.concatenate([u, u], axis=1)
        mlo_m = (jnp.arange(54)[None, :] < 27).astype(jnp.float32)
        bmv = bmr[...]
        ms_lo = jnp.sum(bmv * u2 * mlo_m, axis=1, keepdims=True) + const
        ms_hi = jnp.sum(bmv * u2 * (1.0 - mlo_m), axis=1, keepdims=True) + const
        w2 = jnp.concatenate([w, w], axis=1)
        dlo = (jnp.arange(2 * _D)[None, :] < _D).astype(jnp.float32)
        hs = hsa[...] + hsb[...]
        pos = jnp.maximum(hs + hpa[...] + hpb[...], 0.0) * w2
        neg = jnp.maximum(hs + hna[...] + hnb[...], 0.0) * w2
        opos[...] = jnp.concatenate(
            [jnp.sum(pos * dlo, axis=1, keepdims=True) + ms_lo,
             jnp.sum(pos * (1.0 - dlo), axis=1, keepdims=True) + ms_hi], axis=1)
        oneg[...] = jnp.concatenate(
            [jnp.sum(neg * dlo, axis=1, keepdims=True) + ms_lo,
             jnp.sum(neg * (1.0 - dlo), axis=1, keepdims=True) + ms_hi], axis=1)

    hspec = [
        pl.BlockSpec((_BLK2, 2 * _D), lambda i: (i, 0)),
        pl.BlockSpec((_BLK2, 2 * _D), lambda i: (i + _NB2, 0)),
        pl.BlockSpec((_BLK2, 2 * _D), lambda i: (i + 2 * _NB2, 0)),
    ]
    return pl.pallas_call(
        body,
        grid=(_NB2,),
        in_specs=hspec + hspec + [
            pl.BlockSpec((_BLK2, 54), lambda i: (i, 0)),
            pl.BlockSpec((_D, 27), lambda i: (0, 0)),
            pl.BlockSpec((_D,), lambda i: (0,)),
            pl.BlockSpec((1, _D), lambda i: (0, 0)),
            pl.BlockSpec((1,), lambda i: (0,)),
        ],
        out_specs=[
            pl.BlockSpec((_BLK2, 2), lambda i: (i, 0)),
            pl.BlockSpec((_BLK2, 2), lambda i: (i, 0)),
        ],
        out_shape=[
            jax.ShapeDtypeStruct((_B // 2, 2), jnp.float32),
            jax.ShapeDtypeStruct((_B // 2, 2), jnp.float32),
        ],
    )(h2a, h2a, h2a, h2b, h2b, h2b, bm2, ew, eb, ow, ob)


def kernel(x, src, dst, neg_dst, batch_msg,
           emb0, emb1, emb2, emb3, emb4, emb5, emb6, emb7, emb8, emb9,
           edge_W, edge_b, out_W, out_b):
    xt = x.T
    xcs = [xt[f] for f in range(10)]
    idx_all = jnp.concatenate([src, dst, neg_dst], axis=0)
    ta = [_tc_transpose_table(e.T) for e in (emb1, emb2, emb3, emb4)]
    tb = [_tc_transpose_table(e.T) for e in (emb5, emb6, emb7, emb8, emb9)]
    h2a = _sc_encode_a(xcs[0], *xcs[1:5], idx_all, emb0.reshape(-1), *ta)
    h2b = _sc_encode_b(xcs[0], *xcs[5:10], idx_all, *tb)
    bm2 = batch_msg.reshape(_B // 2, 54)
    op2, on2 = _tc_epilogue(h2a, h2b, bm2, edge_W, edge_b, out_W, out_b)
    return (op2.reshape(_B, 1), on2.reshape(_B, 1))


# speedup vs baseline: 1.1625x; 1.1625x over previous
import functools

import jax
import jax.numpy as jnp
from jax import lax
from jax.experimental import pallas as pl
from jax.experimental.pallas import tpu as pltpu
from jax.experimental.pallas import tpu_sc as plsc

_V = 100000
_D = 64
_B = 16384
_R = 3 * _B
_NC, _NS, _L = 2, 16, 16
_NW = _NC * _NS
_RPW = _R // _NW
_C = 64
_NCH = _RPW // _C

_VP = 102400
_TBLK = 12800


def _tc_transpose_table(et):
    def body(inr, outr):
        outr[:, 0:_D] = jnp.transpose(inr[...])

    return pl.pallas_call(
        body,
        grid=(_VP // _TBLK,),
        in_specs=[pl.BlockSpec((_D, _TBLK), lambda i: (0, i))],
        out_specs=pl.BlockSpec((_TBLK, 2 * _D), lambda i: (i, 0)),
        out_shape=jax.ShapeDtypeStruct((_VP, 2 * _D), jnp.float32),
    )(et)


def _make_sc_encode(nf, mode):
    scratch = [
        pltpu.VMEM((2, _C), jnp.int32),
        pltpu.VMEM((2, nf, _C), jnp.int32),
        pltpu.VMEM((2, _C + _L), jnp.int32),
        pltpu.VMEM((2, nf, _C, 2 * _D), jnp.float32),
        pltpu.VMEM((2, _C // 2, 2 * _D), jnp.float32),
    ]
    if mode == "A":
        scratch.append(pltpu.VMEM((3 * _D,), jnp.float32))
    scratch += [pltpu.SemaphoreType.DMA] * 5

    @functools.partial(
        pl.kernel,
        mesh=plsc.VectorSubcoreMesh(core_axis_name="c", subcore_axis_name="s"),
        out_type=jax.ShapeDtypeStruct((_R // 2, 2 * _D), jnp.float32),
        compiler_params=pltpu.CompilerParams(use_tc_tiling_on_sc=False),
        scratch_types=scratch,
    )
    def enc(*args):
        xc0 = args[0]
        xcs = args[1:1 + nf]
        idx_all = args[1 + nf]
        if mode == "A":
            e0 = args[2 + nf]
            embs = args[3 + nf:3 + 2 * nf]
            (hout, idx_v, cols, tlist, rbuf, hloc, emb0v,
             sem_i, sem_x, sem_t, sem_o0, sem_o1) = args[3 + 2 * nf:]
        else:
            embs = args[2 + nf:2 + 2 * nf]
            (hout, idx_v, cols, tlist, rbuf, hloc,
             sem_i, sem_x, sem_t, sem_o0, sem_o1) = args[2 + 2 * nf:]
        sem_o = (sem_o0, sem_o1)
        wid = lax.axis_index("s") * _NC + lax.axis_index("c")
        if mode == "A":
            pltpu.sync_copy(e0, emb0v)
            e0sl = [[emb0v[pl.ds(t * _D + dv * _L, _L)]
                     for dv in range(_D // _L)] for t in range(3)]
        base_w = wid * _RPW

        def cbase(ci):
            return base_w + ci * _C

        def ix_copy(ci, p):
            return pltpu.make_async_copy(
                idx_all.at[pl.ds(cbase(ci), _C)], idx_v.at[p], sem_i)

        def xcol_copies(ci, p):
            cps = [pltpu.make_async_copy(
                xc0.at[idx_v.at[p]], tlist.at[p, pl.ds(0, _C)], sem_x)]
            cps += [pltpu.make_async_copy(
                xcs[f].at[idx_v.at[p]], cols.at[p, f], sem_x)
                for f in range(nf)]
            return cps

        def table_copies(ci, p):
            return [pltpu.make_async_copy(
                embs[f].at[cols.at[p, f]], rbuf.at[p, f], sem_t)
                for f in range(nf)]

        def hout_copy(ci, p):
            return pltpu.make_async_copy(
                hloc.at[p], hout.at[pl.ds(cbase(ci) // 2, _C // 2)], sem_o[p])

        def combine(p):
            def row(r, c2):
                t = tlist[p, pl.ds(r, _L)][0]
                hrow = r >> 1
                hcol = (r & 1) * _D
                if mode == "A":
                    s0 = jnp.where(t == 0, 1.0, 0.0)
                    s1 = jnp.where(t == 1, 1.0, 0.0)
                    s2 = jnp.where(t == 2, 1.0, 0.0)
                    b0 = lax.broadcast(s0, (_L,))
                    b1 = lax.broadcast(s1, (_L,))
                    b2 = lax.broadcast(s2, (_L,))
                    for dv in range(_D // _L):
                        sl = pl.ds(dv * _L, _L)
                        h0 = (e0sl[0][dv] * b0 + e0sl[1][dv] * b1
                              + e0sl[2][dv] * b2)
                        a = (rbuf[p, 0, r, sl] + rbuf[p, 1, r, sl]
                             + rbuf[p, 2, r, sl] + rbuf[p, 3, r, sl])
                        hloc[p, hrow, pl.ds(hcol + dv * _L, _L)] = h0 + a * b0
                else:
                    s1 = jnp.where(t == 1, 1.0, 0.0)
                    s2 = jnp.where(t == 2, 1.0, 0.0)
                    b1 = lax.broadcast(s1, (_L,))
                    b2 = lax.broadcast(s2, (_L,))
                    for dv in range(_D // _L):
                        sl = pl.ds(dv * _L, _L)
                        b = rbuf[p, 0, r, sl]
                        c = (rbuf[p, 1, r, sl] + rbuf[p, 2, r, sl]
                             + rbuf[p, 3, r, sl] + rbuf[p, 4, r, sl])
                        hloc[p, hrow, pl.ds(hcol + dv * _L, _L)] = (
                            b * b1 + c * b2)
                return c2

            lax.fori_loop(0, _C, row, 0)

        ix_copy(0, 0).start()
        ix_copy(0, 0).wait()
        for cp in xcol_copies(0, 0):
            cp.start()
        for cp in xcol_copies(0, 0):
            cp.wait()
        for cp in table_copies(0, 0):
            cp.start()

        def pair_step(j, carry):
            for p in (0, 1):
                q = 1 - p
                i = 2 * j + p
                nxt = i + 1 < _NCH

                @pl.when(nxt)
                def _(i=i, q=q):
                    ix_copy(i + 1, q).start()

                for cp in table_copies(i, p):
                    cp.wait()

                @pl.when(nxt)
                def _(i=i, q=q):
                    ix_copy(i + 1, q).wait()
                    for cp in xcol_copies(i + 1, q):
                        cp.start()

                @pl.when(i >= 2)
                def _(i=i, p=p):
                    hout_copy(i - 2, p).wait()

                combine(p)

                @pl.when(nxt)
                def _(i=i, q=q):
                    for cp in xcol_copies(i + 1, q):
                        cp.wait()
                    for cp in table_copies(i + 1, q):
                        cp.start()

                hout_copy(i, p).start()
            return carry

        lax.fori_loop(0, _NCH // 2, pair_step, 0)
        hout_copy(_NCH - 2, (_NCH - 2) & 1).wait()
        hout_copy(_NCH - 1, (_NCH - 1) & 1).wait()

    return enc


_sc_encode_a = _make_sc_encode(4, "A")
_sc_encode_b = _make_sc_encode(5, "B")


_BLK2 = 1024
_NB2 = (_B // 2) // _BLK2


def _tc_epilogue(h2a, h2b, bm2, ew, eb, ow, ob):
    def body(hsa, hpa, hna, hsb, hpb, hnb, bmr, ewr, ebr, owr, obr,
             opos, oneg):
        w = owr[...]
        u = jnp.sum(w.T * ewr[...], axis=0, keepdims=True)
        const = jnp.sum(ebr[...] * w[0]) + obr[...][0]
        u2 = jnp.concatenate([u, u], axis=1)
        mlo_m = (jnp.arange(54)[None, :] < 27).astype(jnp.float32)
        bmv = bmr[...]
        ms_lo = jnp.sum(bmv * u2 * mlo_m, axis=1, keepdims=True) + const
        ms_hi = jnp.sum(bmv * u2 * (1.0 - mlo_m), axis=1, keepdims=True) + const
        w2 = jnp.concatenate([w, w], axis=1)
        dlo = (jnp.arange(2 * _D)[None, :] < _D).astype(jnp.float32)
        hs = hsa[...] + hsb[...]
        pos = jnp.maximum(hs + hpa[...] + hpb[...], 0.0) * w2
        neg = jnp.maximum(hs + hna[...] + hnb[...], 0.0) * w2
        opos[...] = jnp.concatenate(
            [jnp.sum(pos * dlo, axis=1, keepdims=True) + ms_lo,
             jnp.sum(pos * (1.0 - dlo), axis=1, keepdims=True) + ms_hi], axis=1)
        oneg[...] = jnp.concatenate(
            [jnp.sum(neg * dlo, axis=1, keepdims=True) + ms_lo,
             jnp.sum(neg * (1.0 - dlo), axis=1, keepdims=True) + ms_hi], axis=1)

    hspec = [
        pl.BlockSpec((_BLK2, 2 * _D), lambda i: (i, 0)),
        pl.BlockSpec((_BLK2, 2 * _D), lambda i: (i + _NB2, 0)),
        pl.BlockSpec((_BLK2, 2 * _D), lambda i: (i + 2 * _NB2, 0)),
    ]
    return pl.pallas_call(
        body,
        grid=(_NB2,),
        in_specs=hspec + hspec + [
            pl.BlockSpec((_BLK2, 54), lambda i: (i, 0)),
            pl.BlockSpec((_D, 27), lambda i: (0, 0)),
            pl.BlockSpec((_D,), lambda i: (0,)),
            pl.BlockSpec((1, _D), lambda i: (0, 0)),
            pl.BlockSpec((1,), lambda i: (0,)),
        ],
        out_specs=[
            pl.BlockSpec((_BLK2, 2), lambda i: (i, 0)),
            pl.BlockSpec((_BLK2, 2), lambda i: (i, 0)),
        ],
        out_shape=[
            jax.ShapeDtypeStruct((_B // 2, 2), jnp.float32),
            jax.ShapeDtypeStruct((_B // 2, 2), jnp.float32),
        ],
    )(h2a, h2a, h2a, h2b, h2b, h2b, bm2, ew, eb, ow, ob)


def kernel(x, src, dst, neg_dst, batch_msg,
           emb0, emb1, emb2, emb3, emb4, emb5, emb6, emb7, emb8, emb9,
           edge_W, edge_b, out_W, out_b):
    xt = x.T
    xcs = [xt[f] for f in range(10)]
    idx_all = jnp.concatenate([src, dst, neg_dst], axis=0)
    ta = [_tc_transpose_table(e.T) for e in (emb1, emb2, emb3, emb4)]
    tb = [_tc_transpose_table(e.T) for e in (emb5, emb6, emb7, emb8, emb9)]
    h2a = _sc_encode_a(xcs[0], *xcs[1:5], idx_all, emb0.reshape(-1), *ta)
    h2b = _sc_encode_b(xcs[0], *xcs[5:10], idx_all, *tb)
    bm2 = batch_msg.reshape(_B // 2, 54)
    op2, on2 = _tc_epilogue(h2a, h2b, bm2, edge_W, edge_b, out_W, out_b)
    return (op2.reshape(_B, 1), on2.reshape(_B, 1))


# speedup vs baseline: 1.2822x; 1.1029x over previous
import functools

import jax
import jax.numpy as jnp
from jax import lax
from jax.experimental import pallas as pl
from jax.experimental.pallas import tpu as pltpu
from jax.experimental.pallas import tpu_sc as plsc

_V = 100000
_D = 64
_B = 16384
_R = 3 * _B
_NC, _NS, _L = 2, 16, 16
_NW = _NC * _NS
_RPW = _R // _NW
_C = 64
_NCH = _RPW // _C

_VP = 102400
_TBLK = 12800


def _tc_transpose_table(et):
    def body(inr, outr):
        outr[:, 0:_D] = jnp.transpose(inr[...])

    return pl.pallas_call(
        body,
        grid=(_VP // _TBLK,),
        in_specs=[pl.BlockSpec((_D, _TBLK), lambda i: (0, i))],
        out_specs=pl.BlockSpec((_TBLK, 2 * _D), lambda i: (i, 0)),
        out_shape=jax.ShapeDtypeStruct((_VP, 2 * _D), jnp.float32),
    )(et)


def _make_sc_encode(nf, mode):
    scratch = [
        pltpu.VMEM((_RPW,), jnp.int32),
        pltpu.VMEM((nf, _RPW), jnp.int32),
        pltpu.VMEM((_RPW + _L,), jnp.int32),
        pltpu.VMEM((2, nf, _C, 2 * _D), jnp.float32),
        pltpu.VMEM((2, _C // 2, 2 * _D), jnp.float32),
    ]
    if mode == "A":
        scratch.append(pltpu.VMEM((3 * _D,), jnp.float32))
    scratch += [pltpu.SemaphoreType.DMA] * 4

    @functools.partial(
        pl.kernel,
        mesh=plsc.VectorSubcoreMesh(core_axis_name="c", subcore_axis_name="s"),
        out_type=jax.ShapeDtypeStruct((_R // 2, 2 * _D), jnp.float32),
        compiler_params=pltpu.CompilerParams(use_tc_tiling_on_sc=False),
        scratch_types=scratch,
    )
    def enc(*args):
        xc0 = args[0]
        xcs = args[1:1 + nf]
        idx_all = args[1 + nf]
        if mode == "A":
            e0 = args[2 + nf]
            embs = args[3 + nf:3 + 2 * nf]
            (hout, idxw, colsw, tl, rbuf, hloc, emb0v,
             sem_x, sem_t, sem_o0, sem_o1) = args[3 + 2 * nf:]
        else:
            embs = args[2 + nf:2 + 2 * nf]
            (hout, idxw, colsw, tl, rbuf, hloc,
             sem_x, sem_t, sem_o0, sem_o1) = args[2 + 2 * nf:]
        sem_o = (sem_o0, sem_o1)
        wid = lax.axis_index("s") * _NC + lax.axis_index("c")
        if mode == "A":
            pltpu.sync_copy(e0, emb0v)
            e0sl = [[emb0v[pl.ds(t * _D + dv * _L, _L)]
                     for dv in range(_D // _L)] for t in range(3)]
        base_w = wid * _RPW
        nsub = _RPW // 128

        def cbase(ci):
            return base_w + ci * _C

        def table_copies(ci, p):
            return [pltpu.make_async_copy(
                embs[f].at[colsw.at[f, pl.ds(ci * _C, _C)]], rbuf.at[p, f],
                sem_t) for f in range(nf)]

        def hout_copy(ci, p):
            return pltpu.make_async_copy(
                hloc.at[p], hout.at[pl.ds(cbase(ci) // 2, _C // 2)], sem_o[p])

        def combine(ci, p):
            def row(r, c2):
                t = tl[pl.ds(ci * _C + r, _L)][0]
                hrow = r >> 1
                hcol = (r & 1) * _D
                if mode == "A":
                    s0 = jnp.where(t == 0, 1.0, 0.0)
                    s1 = jnp.where(t == 1, 1.0, 0.0)
                    s2 = jnp.where(t == 2, 1.0, 0.0)
                    b0 = lax.broadcast(s0, (_L,))
                    b1 = lax.broadcast(s1, (_L,))
                    b2 = lax.broadcast(s2, (_L,))
                    for dv in range(_D // _L):
                        sl = pl.ds(dv * _L, _L)
                        h0 = (e0sl[0][dv] * b0 + e0sl[1][dv] * b1
                              + e0sl[2][dv] * b2)
                        a = (rbuf[p, 0, r, sl] + rbuf[p, 1, r, sl]
                             + rbuf[p, 2, r, sl] + rbuf[p, 3, r, sl])
                        hloc[p, hrow, pl.ds(hcol + dv * _L, _L)] = h0 + a * b0
                else:
                    s1 = jnp.where(t == 1, 1.0, 0.0)
                    s2 = jnp.where(t == 2, 1.0, 0.0)
                    b1 = lax.broadcast(s1, (_L,))
                    b2 = lax.broadcast(s2, (_L,))
                    for dv in range(_D // _L):
                        sl = pl.ds(dv * _L, _L)
                        b = rbuf[p, 0, r, sl]
                        c = (rbuf[p, 1, r, sl] + rbuf[p, 2, r, sl]
                             + rbuf[p, 3, r, sl] + rbuf[p, 4, r, sl])
                        hloc[p, hrow, pl.ds(hcol + dv * _L, _L)] = (
                            b * b1 + c * b2)
                return c2

            lax.fori_loop(0, _C, row, 0)

        pltpu.sync_copy(idx_all.at[pl.ds(base_w, _RPW)], idxw)
        xcps = []
        for j in range(nsub):
            ij = idxw.at[pl.ds(j * 128, 128)]
            xcps.append(pltpu.make_async_copy(
                xc0.at[ij], tl.at[pl.ds(j * 128, 128)], sem_x))
            xcps += [pltpu.make_async_copy(
                xcs[f].at[ij], colsw.at[f, pl.ds(j * 128, 128)],
                sem_x) for f in range(nf)]
        for cp in xcps:
            cp.start()
        for cp in xcps:
            cp.wait()
        for cp in table_copies(0, 0):
            cp.start()

        def pair_step(j, carry):
            for p in (0, 1):
                q = 1 - p
                i = 2 * j + p
                nxt = i + 1 < _NCH

                @pl.when(nxt)
                def _(i=i, q=q):
                    for cp in table_copies(i + 1, q):
                        cp.start()

                for cp in table_copies(i, p):
                    cp.wait()

                @pl.when(i >= 2)
                def _(i=i, p=p):
                    hout_copy(i - 2, p).wait()

                combine(i, p)

                hout_copy(i, p).start()
            return carry

        lax.fori_loop(0, _NCH // 2, pair_step, 0)
        hout_copy(_NCH - 2, (_NCH - 2) & 1).wait()
        hout_copy(_NCH - 1, (_NCH - 1) & 1).wait()

    return enc


_sc_encode_a = _make_sc_encode(4, "A")
_sc_encode_b = _make_sc_encode(5, "B")


_BLK2 = 1024
_NB2 = (_B // 2) // _BLK2


def _tc_epilogue(h2a, h2b, bm2, ew, eb, ow, ob):
    def body(hsa, hpa, hna, hsb, hpb, hnb, bmr, ewr, ebr, owr, obr,
             opos, oneg):
        w = owr[...]
        u = jnp.sum(w.T * ewr[...], axis=0, keepdims=True)
        const = jnp.sum(ebr[...] * w[0]) + obr[...][0]
        u2 = jnp.concatenate([u, u], axis=1)
        mlo_m = (jnp.arange(54)[None, :] < 27).astype(jnp.float32)
        bmv = bmr[...]
        ms_lo = jnp.sum(bmv * u2 * mlo_m, axis=1, keepdims=True) + const
        ms_hi = jnp.sum(bmv * u2 * (1.0 - mlo_m), axis=1, keepdims=True) + const
        w2 = jnp.concatenate([w, w], axis=1)
        dlo = (jnp.arange(2 * _D)[None, :] < _D).astype(jnp.float32)
        hs = hsa[...] + hsb[...]
        pos = jnp.maximum(hs + hpa[...] + hpb[...], 0.0) * w2
        neg = jnp.maximum(hs + hna[...] + hnb[...], 0.0) * w2
        opos[...] = jnp.concatenate(
            [jnp.sum(pos * dlo, axis=1, keepdims=True) + ms_lo,
             jnp.sum(pos * (1.0 - dlo), axis=1, keepdims=True) + ms_hi], axis=1)
        oneg[...] = jnp.concatenate(
            [jnp.sum(neg * dlo, axis=1, keepdims=True) + ms_lo,
             jnp.sum(neg * (1.0 - dlo), axis=1, keepdims=True) + ms_hi], axis=1)

    hspec = [
        pl.BlockSpec((_BLK2, 2 * _D), lambda i: (i, 0)),
        pl.BlockSpec((_BLK2, 2 * _D), lambda i: (i + _NB2, 0)),
        pl.BlockSpec((_BLK2, 2 * _D), lambda i: (i + 2 * _NB2, 0)),
    ]
    return pl.pallas_call(
        body,
        grid=(_NB2,),
        in_specs=hspec + hspec + [
            pl.BlockSpec((_BLK2, 54), lambda i: (i, 0)),
            pl.BlockSpec((_D, 27), lambda i: (0, 0)),
            pl.BlockSpec((_D,), lambda i: (0,)),
            pl.BlockSpec((1, _D), lambda i: (0, 0)),
            pl.BlockSpec((1,), lambda i: (0,)),
        ],
        out_specs=[
            pl.BlockSpec((_BLK2, 2), lambda i: (i, 0)),
            pl.BlockSpec((_BLK2, 2), lambda i: (i, 0)),
        ],
        out_shape=[
            jax.ShapeDtypeStruct((_B // 2, 2), jnp.float32),
            jax.ShapeDtypeStruct((_B // 2, 2), jnp.float32),
        ],
    )(h2a, h2a, h2a, h2b, h2b, h2b, bm2, ew, eb, ow, ob)


def kernel(x, src, dst, neg_dst, batch_msg,
           emb0, emb1, emb2, emb3, emb4, emb5, emb6, emb7, emb8, emb9,
           edge_W, edge_b, out_W, out_b):
    xt = x.T
    xcs = [xt[f] for f in range(10)]
    idx_all = jnp.concatenate([src, dst, neg_dst], axis=0)
    ta = [_tc_transpose_table(e.T) for e in (emb1, emb2, emb3, emb4)]
    tb = [_tc_transpose_table(e.T) for e in (emb5, emb6, emb7, emb8, emb9)]
    h2a = _sc_encode_a(xcs[0], *xcs[1:5], idx_all, emb0.reshape(-1), *ta)
    h2b = _sc_encode_b(xcs[0], *xcs[5:10], idx_all, *tb)
    bm2 = batch_msg.reshape(_B // 2, 54)
    op2, on2 = _tc_epilogue(h2a, h2b, bm2, edge_W, edge_b, out_W, out_b)
    return (op2.reshape(_B, 1), on2.reshape(_B, 1))
